# Initial kernel scaffold; baseline (speedup 1.0000x reference)
import jax
import jax.numpy as jnp
from jax.experimental import pallas as pl


def kernel(QR, symbols, W, b):
    raise NotImplementedError("write your pallas kernel here")


# speedup vs baseline: 1.7253x; 1.7253x over previous
import functools

import jax
import jax.numpy as jnp
from jax import lax
from jax.experimental import pallas as pl
from jax.experimental.pallas import tpu as pltpu
from jax.experimental.pallas import tpu_sc as plsc

OUT_DIM = 512
NC, NS = 2, 16
NW = NC * NS
CHUNK = 64


def _table_body(sym_ref, w_ref, b_ref, out_ref):
    out_ref[:] = (
        jnp.dot(sym_ref[:], w_ref[:], preferred_element_type=jnp.float32)
        + b_ref[:]
    )


def _make_table(symbols, W, b):
    vocab = symbols.shape[0]
    return pl.pallas_call(
        _table_body,
        out_shape=jax.ShapeDtypeStruct((vocab, OUT_DIM), jnp.float32),
    )(symbols, W, b.reshape(1, OUT_DIM))


def _gather_body(n_chunks, table, idx, out, idx_v, buf_a, buf_b, sem_a, sem_b):
    wid = lax.axis_index("s") * NC + lax.axis_index("c")
    base = wid * (n_chunks * CHUNK)
    pltpu.sync_copy(idx.at[wid], idx_v)

    def gather(c, buf, sem):
        pltpu.async_copy(table.at[idx_v.at[c]], buf, sem)

    def gwait(buf, sem):
        pltpu.make_async_copy(table.at[idx_v.at[0]], buf, sem).wait()

    def put(c, buf):
        pltpu.sync_copy(buf, out.at[pl.ds(base + c * CHUNK, CHUNK)])

    gather(0, buf_a, sem_a)

    def body(i, carry):
        j = 2 * i
        gwait(buf_a, sem_a)
        gather(j + 1, buf_b, sem_b)
        put(j, buf_a)
        gwait(buf_b, sem_b)
        gather(j + 2, buf_a, sem_a)
        put(j + 1, buf_b)
        return carry

    lax.fori_loop(0, n_chunks // 2 - 1, body, 0)

    j = n_chunks - 2
    gwait(buf_a, sem_a)
    gather(j + 1, buf_b, sem_b)
    put(j, buf_a)
    gwait(buf_b, sem_b)
    put(j + 1, buf_b)


def _gather_rows(table, idx3d, n_chunks):
    rows = NW * n_chunks * CHUNK
    mesh = plsc.VectorSubcoreMesh(core_axis_name="c", subcore_axis_name="s")
    k = pl.kernel(
        functools.partial(_gather_body, n_chunks),
        mesh=mesh,
        out_type=jax.ShapeDtypeStruct((rows, OUT_DIM), jnp.float32),
        scratch_types=[
            pltpu.VMEM((n_chunks, CHUNK), jnp.int32),
            pltpu.VMEM((CHUNK, OUT_DIM), jnp.float32),
            pltpu.VMEM((CHUNK, OUT_DIM), jnp.float32),
            pltpu.SemaphoreType.DMA,
            pltpu.SemaphoreType.DMA,
        ],
    )
    return k(table, idx3d)


def kernel(QR, symbols, W, b):
    batch, seq = QR.shape
    rows = batch * seq
    n_chunks = rows // (NW * CHUNK)
    assert rows == NW * n_chunks * CHUNK and n_chunks % 2 == 0

    table = _make_table(symbols, W, b)
    idx3d = QR.astype(jnp.int32).reshape(NW, n_chunks, CHUNK)
    out = _gather_rows(table, idx3d, n_chunks)
    return out.reshape(batch, seq, OUT_DIM)
